# Initial kernel scaffold; baseline (speedup 1.0000x reference)
import jax
import jax.numpy as jnp
from jax.experimental import pallas as pl


def kernel(x, token_table, pos_table):
    raise NotImplementedError("write your pallas kernel here")


# speedup vs baseline: 2.1382x; 2.1382x over previous
import functools

import jax
import jax.numpy as jnp
from jax import lax
from jax.experimental import pallas as pl
from jax.experimental.pallas import tpu as pltpu
from jax.experimental.pallas import tpu_sc as plsc

VOCAB = 100000
SEQ_LEN = 200
HIDDEN = 64
BATCH = 4096

NC = 2
NS = 16
NW = NC * NS

ROWS = BATCH * SEQ_LEN
ROWS_PER_W = ROWS // NW
CHUNK = 128
N_CHUNKS = ROWS_PER_W // CHUNK
HVEC = HIDDEN // 16


def _body(x_hbm, tok_hbm, pos_hbm, out_hbm, idx_v, pe_v, rows_v, sem):
    wid = lax.axis_index("s") * NC + lax.axis_index("c")
    wbase = wid * ROWS_PER_W

    pltpu.sync_copy(pos_hbm, pe_v)

    def chunk_body(g, _):
        base = wbase + g * CHUNK
        pltpu.sync_copy(x_hbm.at[pl.ds(base, CHUNK)], idx_v)
        pltpu.async_copy(tok_hbm.at[idx_v], rows_v, sem).wait()

        base_pos = lax.rem(base, SEQ_LEN)

        def row_body(r, _):
            p = base_pos + r
            p = jnp.where(p >= SEQ_LEN, p - SEQ_LEN, p)
            for v in range(HVEC):
                sl = pl.ds(v * 16, 16)
                rows_v[r, sl] = rows_v[r, sl] + pe_v[p, sl]
            return 0

        lax.fori_loop(0, CHUNK, row_body, 0)
        pltpu.sync_copy(rows_v, out_hbm.at[pl.ds(base, CHUNK)])
        return 0

    lax.fori_loop(0, N_CHUNKS, chunk_body, 0)


@jax.jit
def _encode(x_flat, token_table, pos_table):
    mesh = plsc.VectorSubcoreMesh(core_axis_name="c", subcore_axis_name="s")
    return pl.kernel(
        _body,
        out_type=jax.ShapeDtypeStruct((ROWS, HIDDEN), jnp.float32),
        mesh=mesh,
        scratch_types=[
            pltpu.VMEM((CHUNK,), jnp.int32),
            pltpu.VMEM((SEQ_LEN, HIDDEN), jnp.float32),
            pltpu.VMEM((CHUNK, HIDDEN), jnp.float32),
            pltpu.SemaphoreType.DMA,
        ],
        compiler_params=pltpu.CompilerParams(use_tc_tiling_on_sc=False),
    )(x_flat, token_table, pos_table)


def kernel(x, token_table, pos_table):
    x_flat = x.reshape(-1).astype(jnp.int32)
    out = _encode(x_flat, token_table, pos_table)
    return out.reshape(x.shape[0], x.shape[1], HIDDEN)


# speedup vs baseline: 5.1638x; 2.4150x over previous
import jax
import jax.numpy as jnp
from jax import lax
from jax.experimental import pallas as pl
from jax.experimental.pallas import tpu as pltpu
from jax.experimental.pallas import tpu_sc as plsc

VOCAB = 100000
SEQ_LEN = 200
HIDDEN = 64
BATCH = 4096

NC = 2
NS = 16
NW = NC * NS

ROWS = BATCH * SEQ_LEN
ROWS_PER_W = ROWS // NW
SEQ_PER_CHUNK = 2
CHUNK = SEQ_PER_CHUNK * SEQ_LEN
N_CHUNKS = ROWS_PER_W // CHUNK
N_BODIES = N_CHUNKS // 2


def _body(x_hbm, tok_hbm, pos_hbm, out_hbm,
          idx4_v, pe_img_v, rows_a, rows_b,
          isem_e, isem_o, gsem_a, gsem_b, osem_a, osem_b):
    wid = lax.axis_index("s") * NC + lax.axis_index("c")
    wbase = wid * ROWS_PER_W

    def i_start(c, isem):
        base = wbase + c * CHUNK
        s4 = lax.rem(c, 4)
        pltpu.async_copy(x_hbm.at[pl.ds(base, CHUNK)], idx4_v.at[s4], isem)

    def issue(c, rows_v, gsem, osem, isem):
        base = wbase + c * CHUNK
        s4 = lax.rem(c, 4)
        pltpu.make_async_copy(
            x_hbm.at[pl.ds(base, CHUNK)], idx4_v.at[s4], isem).wait()

        @pl.when(c >= 2)
        def _():
            pltpu.make_async_copy(
                rows_v, out_hbm.at[pl.ds(base, CHUNK)], osem).wait()

        def pe_row(r, _):
            for v in range(HIDDEN // 16):
                sl = pl.ds(v * 16, 16)
                pe = pe_img_v[r, sl]
                for k in range(SEQ_PER_CHUNK):
                    rows_v[k * SEQ_LEN + r, sl] = pe
            return 0

        lax.fori_loop(0, SEQ_LEN, pe_row, 0)
        pltpu.async_copy(tok_hbm.at[idx4_v.at[s4]], rows_v, gsem, add=True)

    def complete(c, rows_v, gsem, osem, isem_c2):
        base = wbase + c * CHUNK
        s4 = lax.rem(c, 4)
        pltpu.make_async_copy(
            tok_hbm.at[idx4_v.at[s4]], rows_v, gsem).wait()
        pltpu.async_copy(rows_v, out_hbm.at[pl.ds(base, CHUNK)], osem)

        @pl.when(c + 2 < N_CHUNKS)
        def _():
            i_start(c + 2, isem_c2)

    pltpu.sync_copy(pos_hbm, pe_img_v)

    i_start(jnp.int32(0), isem_e)
    i_start(jnp.int32(1), isem_o)

    def loop_body(t, _):
        c0 = 2 * t
        c1 = c0 + 1
        issue(c0, rows_a, gsem_a, osem_a, isem_e)

        @pl.when(t > 0)
        def _():
            complete(c1 - 2, rows_b, gsem_b, osem_b, isem_o)

        issue(c1, rows_b, gsem_b, osem_b, isem_o)
        complete(c0, rows_a, gsem_a, osem_a, isem_e)
        return 0

    lax.fori_loop(0, N_BODIES, loop_body, 0)

    last = jnp.int32(N_CHUNKS - 1)
    complete(last, rows_b, gsem_b, osem_b, isem_o)
    pltpu.make_async_copy(
        rows_a, out_hbm.at[pl.ds(wbase, CHUNK)], osem_a).wait()
    pltpu.make_async_copy(
        rows_b, out_hbm.at[pl.ds(wbase, CHUNK)], osem_b).wait()


@jax.jit
def _encode(x_flat, token_table, pos_table):
    mesh = plsc.VectorSubcoreMesh(core_axis_name="c", subcore_axis_name="s")
    return pl.kernel(
        _body,
        out_type=jax.ShapeDtypeStruct((ROWS, HIDDEN), jnp.float32),
        mesh=mesh,
        scratch_types=[
            pltpu.VMEM((4, CHUNK), jnp.int32),
            pltpu.VMEM((SEQ_LEN, HIDDEN), jnp.float32),
            pltpu.VMEM((CHUNK, HIDDEN), jnp.float32),
            pltpu.VMEM((CHUNK, HIDDEN), jnp.float32),
            pltpu.SemaphoreType.DMA,
            pltpu.SemaphoreType.DMA,
            pltpu.SemaphoreType.DMA,
            pltpu.SemaphoreType.DMA,
            pltpu.SemaphoreType.DMA,
            pltpu.SemaphoreType.DMA,
        ],
        compiler_params=pltpu.CompilerParams(use_tc_tiling_on_sc=False),
    )(x_flat, token_table, pos_table)


def kernel(x, token_table, pos_table):
    x_flat = x.reshape(-1).astype(jnp.int32)
    out = _encode(x_flat, token_table, pos_table)
    return out.reshape(x.shape[0], x.shape[1], HIDDEN)
